# baseline (speedup 1.0000x reference)
import functools

import jax
import jax.numpy as jnp
from jax import lax
from jax.experimental import pallas as pl
from jax.experimental.pallas import tpu as pltpu
from jax.experimental.pallas import tpu_sc as plsc

D = 64
NC = 2
NS = 16
L = 16
NW = NC * NS
B = 16384
B_PER_W = B // NW
CHUNK = 128
NCHUNK = B_PER_W // CHUNK
GPC = CHUNK // L

_PARAMS = dict(needs_layout_passes=False, use_tc_tiling_on_sc=False,
               disable_bounds_checks=True)


def _mesh():
    return plsc.VectorSubcoreMesh(core_axis_name="c", subcore_axis_name="s",
                                  num_cores=NC, num_subcores=NS)


def _log_sigmoid(x):
    z = jnp.exp(-jnp.abs(x))
    t = z / (z + 2.0)
    t2 = t * t
    p = 1.0 / 9.0 + t2 * (1.0 / 11.0)
    p = 1.0 / 7.0 + t2 * p
    p = 1.0 / 5.0 + t2 * p
    p = 1.0 / 3.0 + t2 * p
    p = 1.0 + t2 * p
    return jnp.minimum(x, 0.0) - 2.0 * t * p


@functools.cache
def _gather_sc():
    @functools.partial(
        pl.kernel,
        mesh=_mesh(),
        out_type=jax.ShapeDtypeStruct((B, D), jnp.float32),
        compiler_params=pltpu.CompilerParams(**_PARAMS),
        scratch_types=[
            pltpu.VMEM((NCHUNK, CHUNK), jnp.int32),
            pltpu.VMEM((NCHUNK, CHUNK, D), jnp.float32),
            pltpu.SemaphoreType.DMA,
        ],
    )
    def body(pos2, emb, out, idx, rows, sem):
        wid = lax.axis_index("s") * NC + lax.axis_index("c")
        pltpu.sync_copy(pos2.at[pl.ds(wid * NCHUNK, NCHUNK)], idx)
        copies = [pltpu.async_copy(emb.at[idx.at[j]], rows.at[j], sem)
                  for j in range(NCHUNK)]
        for j in range(NCHUNK):
            copies[j].wait()
            pltpu.sync_copy(
                rows.at[j],
                out.at[pl.ds(wid * B_PER_W + j * CHUNK, CHUNK)])

    return body


@functools.cache
def _loss_sc():
    @functools.partial(
        pl.kernel,
        mesh=_mesh(),
        out_type=jax.ShapeDtypeStruct((NW, L), jnp.float32),
        compiler_params=pltpu.CompilerParams(**_PARAMS),
        scratch_types=[
            pltpu.VMEM((B_PER_W, D), jnp.float32),
            pltpu.VMEM((B_PER_W, D), jnp.float32),
            pltpu.VMEM((L,), jnp.float32),
            pltpu.SemaphoreType.DMA,
            pltpu.SemaphoreType.DMA,
        ],
    )
    def body(u_all, v_all, out, u_rows, v_rows, out_v, usem, vsem):
        wid = lax.axis_index("s") * NC + lax.axis_index("c")
        base = wid * B_PER_W
        cu = pltpu.async_copy(u_all.at[pl.ds(base, B_PER_W)], u_rows, usem)
        cv = pltpu.async_copy(v_all.at[pl.ds(base, B_PER_W)], v_rows, vsem)
        cu.wait()
        cv.wait()

        lane = lax.iota(jnp.int32, L)

        def group_body(g, acc):
            score = jnp.zeros((L,), jnp.float32)
            for r in range(L):
                row = g * L + r
                prod = u_rows[row, pl.ds(0, L)] * v_rows[row, pl.ds(0, L)]
                for c in range(1, D // L):
                    prod = prod + (u_rows[row, pl.ds(c * L, L)]
                                   * v_rows[row, pl.ds(c * L, L)])
                total = jnp.sum(prod)
                score = jnp.where(lane == r, total, score)
            return acc + _log_sigmoid(score)

        acc = lax.fori_loop(0, B_PER_W // L, group_body,
                            jnp.zeros((L,), jnp.float32))
        out_v[...] = acc
        pltpu.sync_copy(out_v, out.at[wid])

    return body


def kernel(u_pos, v_pos, batch_size, u_embeddings, v_embeddings):
    gather = _gather_sc()
    u_rows = gather(u_pos.reshape(B // CHUNK, CHUNK), u_embeddings)
    v_rows = gather(v_pos.reshape(B // CHUNK, CHUNK), v_embeddings)
    partials = _loss_sc()(u_rows, v_rows)
    return -jnp.sum(partials) / batch_size


# speedup vs baseline: 1.0421x; 1.0421x over previous
import functools

import jax
import jax.numpy as jnp
from jax import lax
from jax.experimental import pallas as pl
from jax.experimental.pallas import tpu as pltpu
from jax.experimental.pallas import tpu_sc as plsc

D = 64
NC = 2
NS = 16
L = 16
NW = NC * NS
B = 16384
B_PER_W = B // NW
CHUNK = 128
NCHUNK = B_PER_W // CHUNK
GPC = CHUNK // L


def _log_sigmoid(x):
    z = jnp.exp(-jnp.abs(x))
    t = z / (z + 2.0)
    t2 = t * t
    p = 1.0 / 9.0 + t2 * (1.0 / 11.0)
    p = 1.0 / 7.0 + t2 * p
    p = 1.0 / 5.0 + t2 * p
    p = 1.0 / 3.0 + t2 * p
    p = 1.0 + t2 * p
    return jnp.minimum(x, 0.0) - 2.0 * t * p


@functools.cache
def _skipgram_sc():
    @functools.partial(
        pl.kernel,
        mesh=plsc.VectorSubcoreMesh(core_axis_name="c", subcore_axis_name="s",
                                    num_cores=NC, num_subcores=NS),
        out_type=jax.ShapeDtypeStruct((NW, L), jnp.float32),
        compiler_params=pltpu.CompilerParams(needs_layout_passes=False,
                                             use_tc_tiling_on_sc=False,
                                             disable_bounds_checks=True),
        scratch_types=[
            pltpu.VMEM((NCHUNK, CHUNK), jnp.int32),
            pltpu.VMEM((NCHUNK, CHUNK), jnp.int32),
            pltpu.VMEM((NCHUNK, CHUNK, D), jnp.float32),
            pltpu.VMEM((NCHUNK, CHUNK, D), jnp.float32),
            pltpu.VMEM((L,), jnp.float32),
            pltpu.SemaphoreType.DMA,
            pltpu.SemaphoreType.DMA,
            pltpu.SemaphoreType.DMA,
            pltpu.SemaphoreType.DMA,
        ],
    )
    def body(u_pos2, v_pos2, u_emb, v_emb, out, u_idx, v_idx,
             u_rows, v_rows, out_v, sem0, sem1, sem2, sem3):
        wid = lax.axis_index("s") * NC + lax.axis_index("c")
        ib = wid * NCHUNK
        pltpu.sync_copy(u_pos2.at[pl.ds(ib, NCHUNK)], u_idx)
        pltpu.sync_copy(v_pos2.at[pl.ds(ib, NCHUNK)], v_idx)
        sems = [sem0, sem1, sem2, sem3]
        copies = []
        for j in range(NCHUNK):
            copies.append(
                (pltpu.async_copy(u_emb.at[u_idx.at[j]], u_rows.at[j], sems[j]),
                 pltpu.async_copy(v_emb.at[v_idx.at[j]], v_rows.at[j], sems[j])))

        lane = lax.iota(jnp.int32, L)
        acc = jnp.zeros((L,), jnp.float32)
        for j in range(NCHUNK):
            cu, cv = copies[j]
            cu.wait()
            cv.wait()

            def group_body(g, acc, j=j):
                score = jnp.zeros((L,), jnp.float32)
                for r in range(L):
                    row = g * L + r
                    prod = (u_rows[j, row, pl.ds(0, L)]
                            * v_rows[j, row, pl.ds(0, L)])
                    for c in range(1, D // L):
                        prod = prod + (u_rows[j, row, pl.ds(c * L, L)]
                                       * v_rows[j, row, pl.ds(c * L, L)])
                    total = jnp.sum(prod)
                    score = jnp.where(lane == r, total, score)
                return acc + _log_sigmoid(score)

            acc = lax.fori_loop(0, GPC, group_body, acc)

        out_v[...] = acc
        pltpu.sync_copy(out_v, out.at[wid])

    return body


def kernel(u_pos, v_pos, batch_size, u_embeddings, v_embeddings):
    u2 = u_pos.reshape(B // CHUNK, CHUNK)
    v2 = v_pos.reshape(B // CHUNK, CHUNK)
    partials = _skipgram_sc()(u2, v2, u_embeddings, v_embeddings)
    return -jnp.sum(partials) / batch_size
